# baseline (speedup 1.0000x reference)
import jax
import jax.numpy as jnp
from jax import lax
from jax.experimental import pallas as pl
from jax.experimental.pallas import tpu as pltpu
from jax.experimental.pallas import tpu_sc as plsc

_LANES = 16
_NC = 2
_NS = 16


def _make_body(n, span, num_cores):
    def _body(e_hbm, idx_hbm, sh_hbm, sc_hbm, out_hbm,
              sh_v, sc_v, idx_v, e_v, o_v, sem):
        wid = lax.axis_index("s") * num_cores + lax.axis_index("c")
        base = jnp.minimum(wid * span, n - span)
        half = span // 2
        c1 = pltpu.async_copy(sh_hbm, sh_v, sem)
        c2 = pltpu.async_copy(sc_hbm, sc_v, sem)
        c3 = pltpu.async_copy(e_hbm.at[pl.ds(base, span)], e_v, sem)
        c4 = pltpu.async_copy(idx_hbm.at[pl.ds(base, span)], idx_v, sem)
        c1.wait(); c2.wait(); c3.wait(); c4.wait()

        @plsc.parallel_loop(0, half // _LANES, unroll=7)
        def _(j):
            sl = pl.ds(j * _LANES, _LANES)
            iv = idx_v[sl]
            sv = plsc.load_gather(sh_v, [iv])
            cv = plsc.load_gather(sc_v, [iv])
            o_v[sl] = e_v[sl] * cv + sv

        co = pltpu.async_copy(
            o_v.at[pl.ds(0, half)], out_hbm.at[pl.ds(base, half)], sem
        )

        @plsc.parallel_loop(half // _LANES, span // _LANES, unroll=7)
        def _(j):
            sl = pl.ds(j * _LANES, _LANES)
            iv = idx_v[sl]
            sv = plsc.load_gather(sh_v, [iv])
            cv = plsc.load_gather(sc_v, [iv])
            o_v[sl] = e_v[sl] * cv + sv

        co.wait()
        pltpu.sync_copy(
            o_v.at[pl.ds(half, span - half)],
            out_hbm.at[pl.ds(base + half, span - half)],
        )

    return _body


def kernel(energy, species_idx, shifts, scales):
    n = energy.shape[0]
    n_workers = _NC * _NS
    span = -(-n // (n_workers * _LANES)) * _LANES
    assert span % 8 == 0 and (n - span) % 8 == 0 and span <= n
    n_types = shifts.shape[0]
    mesh = plsc.VectorSubcoreMesh(
        core_axis_name="c", subcore_axis_name="s", num_cores=_NC, num_subcores=_NS
    )
    run = pl.kernel(
        _make_body(n, span, mesh.num_cores),
        out_type=jax.ShapeDtypeStruct((n,), jnp.float32),
        mesh=mesh,
        compiler_params=pltpu.CompilerParams(needs_layout_passes=False),
        scratch_types=[
            pltpu.VMEM((n_types,), jnp.float32),
            pltpu.VMEM((n_types,), jnp.float32),
            pltpu.VMEM((span,), jnp.int32),
            pltpu.VMEM((span,), jnp.float32),
            pltpu.VMEM((span,), jnp.float32),
            pltpu.SemaphoreType.DMA,
        ],
    )
    return run(energy.reshape(n), species_idx, shifts, scales).reshape(n, 1)


# speedup vs baseline: 1.0067x; 1.0067x over previous
import jax
import jax.numpy as jnp
from jax import lax
from jax.experimental import pallas as pl
from jax.experimental.pallas import tpu as pltpu
from jax.experimental.pallas import tpu_sc as plsc

_LANES = 16
_NC = 2
_NS = 16


def _make_body(n, span, num_cores):
    def _body(e_hbm, idx_hbm, sh_hbm, sc_hbm, out_hbm,
              sh_v, sc_v, idx_v, e_v, o_v, sem):
        wid = lax.axis_index("s") * num_cores + lax.axis_index("c")
        base = jnp.minimum(wid * span, n - span)
        half = span // 2
        c1 = pltpu.async_copy(sh_hbm, sh_v, sem)
        c2 = pltpu.async_copy(sc_hbm, sc_v, sem)
        c3 = pltpu.async_copy(e_hbm.at[pl.ds(base, span)], e_v, sem)
        c4 = pltpu.async_copy(idx_hbm.at[pl.ds(base, span)], idx_v, sem)
        c1.wait(); c2.wait(); c3.wait(); c4.wait()

        @plsc.parallel_loop(0, span // _LANES, unroll=7)
        def _(j):
            sl = pl.ds(j * _LANES, _LANES)
            iv = idx_v[sl]
            sv = plsc.load_gather(sh_v, [iv])
            cv = plsc.load_gather(sc_v, [iv])
            o_v[sl] = e_v[sl] * cv + sv

        pltpu.sync_copy(o_v, out_hbm.at[pl.ds(base, span)])

    return _body


def kernel(energy, species_idx, shifts, scales):
    n = energy.shape[0]
    n_workers = _NC * _NS
    span = -(-n // (n_workers * _LANES)) * _LANES
    assert span % 8 == 0 and (n - span) % 8 == 0 and span <= n
    n_types = shifts.shape[0]
    mesh = plsc.VectorSubcoreMesh(
        core_axis_name="c", subcore_axis_name="s", num_cores=_NC, num_subcores=_NS
    )
    run = pl.kernel(
        _make_body(n, span, mesh.num_cores),
        out_type=jax.ShapeDtypeStruct((n,), jnp.float32),
        mesh=mesh,
        compiler_params=pltpu.CompilerParams(needs_layout_passes=False),
        scratch_types=[
            pltpu.VMEM((n_types,), jnp.float32),
            pltpu.VMEM((n_types,), jnp.float32),
            pltpu.VMEM((span,), jnp.int32),
            pltpu.VMEM((span,), jnp.float32),
            pltpu.VMEM((span,), jnp.float32),
            pltpu.SemaphoreType.DMA,
        ],
    )
    return run(energy.reshape(n), species_idx, shifts, scales).reshape(n, 1)
